# baseline (speedup 1.0000x reference)
import functools

import jax
import jax.numpy as jnp
from jax import lax
from jax.experimental import pallas as pl
from jax.experimental.pallas import tpu as pltpu
from jax.experimental.pallas import tpu_sc as plsc

_K = 256
_L = 16
_ROWS = 1024
_ROW = 8192
_NV = _ROW // _L
_NW = 32
_RPW = _ROWS // _NW
_SIGN = -2147483648


def _batcher_pairs(n):
    pairs = []

    def merge(lo, cnt, r):
        step = r * 2
        if step < cnt:
            merge(lo, cnt, step)
            merge(lo + r, cnt, step)
            for i in range(lo + r, lo + cnt - r, step):
                pairs.append((i, i + r))
        else:
            pairs.append((lo, lo + r))

    def sort(lo, cnt):
        if cnt > 1:
            m = cnt // 2
            sort(lo, m)
            sort(lo + m, m)
            merge(lo, cnt, 1)

    sort(0, n)
    return pairs


_PAIRS16 = _batcher_pairs(16)


def _vsort_desc(v):
    ks, _ = plsc.sort_key_val(v, v, descending=True)
    return ks


def _extract(vec, idx, lanes):
    return jnp.sum(jnp.where(lanes == idx, vec, 0))


def _suffix(v):
    return lax.rev(plsc.cumsum(lax.rev(v, (0,))), (0,))


def _find_bucket256(block_vec, bs_list, k, lanes):
    sfx = [None] * 16
    run = jnp.int32(0)
    for i in range(15, -1, -1):
        run = run + bs_list[i]
        sfx[i] = run
    bidx = jnp.int32(0)
    sfx_sel = sfx[0]
    bs_sel = bs_list[0]
    for i in range(1, 16):
        c = sfx[i] >= k
        bidx = jnp.where(c, jnp.int32(i), bidx)
        sfx_sel = jnp.where(c, sfx[i], sfx_sel)
        bs_sel = jnp.where(c, bs_list[i], bs_sel)
    bt = block_vec(bidx)
    s_vec = _suffix(bt) + (sfx_sel - bs_sel)
    mask = s_vec >= k
    nge = plsc.all_reduce_population_count(mask)[0]
    boff = nge - 1
    s_b = _extract(s_vec, boff, lanes)
    t_b = _extract(bt, boff, lanes)
    return bidx * 16 + boff, s_b - t_b


def _make_kernel():
    mesh = plsc.VectorSubcoreMesh(core_axis_name="c", subcore_axis_name="s")

    @functools.partial(
        pl.kernel,
        mesh=mesh,
        compiler_params=pltpu.CompilerParams(needs_layout_passes=False),
        out_type=jax.ShapeDtypeStruct((_ROWS, _K * 2), jnp.int32),
        scratch_types=[
            pltpu.VMEM((_ROW,), jnp.int32),
            pltpu.VMEM((_ROW,), jnp.int32),
            pltpu.VMEM((4 * 256 * _L,), jnp.int32),
            pltpu.VMEM((256,), jnp.int32),
            pltpu.VMEM((2 * 16 * _L,), jnp.int32),
            pltpu.VMEM((_ROW,), jnp.int32),
            pltpu.VMEM((_ROW,), jnp.int32),
            pltpu.VMEM((_ROW,), jnp.int32),
            pltpu.VMEM((256 * _L,), jnp.int32),
            pltpu.VMEM((256 * _L,), jnp.int32),
            pltpu.VMEM((_K + _L,), jnp.int32),
            pltpu.VMEM((_K + _L,), jnp.int32),
            pltpu.VMEM((2 * _K,), jnp.int32),
        ],
    )
    def minmax(in_hbm, out_hbm, rowf, keys, hist, totb, hist16, tie_t, tie_b,
               tie_s, conf_t, conf_b, gtop, gbot, outv):
        wid = lax.axis_index("s") * 2 + lax.axis_index("c")
        lanes = lax.iota(jnp.int32, 16)
        ones = jnp.ones((16,), jnp.int32)
        zeros = jnp.zeros((16,), jnp.int32)
        full_mask = ones > 0

        def totals256():
            def tb(i, _):
                acc = zeros
                for u in range(16):
                    off = (i * 16 + u) * 16
                    v = zeros
                    for r in range(4):
                        v = v + hist[pl.ds(r * 4096 + off, 16)]
                        hist[pl.ds(r * 4096 + off, 16)] = zeros
                    acc = jnp.where(lanes == u, jnp.sum(v), acc)
                totb[pl.ds(i * 16, 16)] = acc
                return 0

            lax.fori_loop(0, 16, tb, 0)
            return [jnp.sum(totb[pl.ds(i * 16, 16)]) for i in range(16)]

        def refine_collect(tie_a, cnt_vec, b1, k2, gref, n0, invert):
            k_cur = k2
            t_acc = b1 ^ 0x80
            cur, other = tie_a, tie_s
            cnt_cur = cnt_vec
            n = n0
            for shift in (20, 16, 12, 8, 4, 0):
                jm2 = (jnp.max(cnt_cur) + 1) // 2

                def hb(j2, _, cur=cur, cnt=cnt_cur, shift=shift):
                    for u in range(2):
                        j = j2 * 2 + u
                        kv = cur[pl.ds(j * 16, 16)]
                        mk = j < cnt
                        dg = lax.shift_right_logical(kv, shift) & 15
                        plsc.addupdate_scatter(
                            hist16, [u * 256 + dg * 16 + lanes], ones,
                            mask=mk)
                    return 0

                lax.fori_loop(0, jm2, hb, 0)

                def tl(u, acc):
                    v = hist16[pl.ds(u * 16, 16)] + hist16[
                        pl.ds(256 + u * 16, 16)]
                    hist16[pl.ds(u * 16, 16)] = zeros
                    hist16[pl.ds(256 + u * 16, 16)] = zeros
                    return jnp.where(lanes == u, jnp.sum(v), acc)

                tot16 = lax.fori_loop(0, 16, tl, zeros)
                s_vec = _suffix(tot16)
                mask = s_vec >= k_cur
                nge = plsc.all_reduce_population_count(mask)[0]
                b = nge - 1
                s_b = _extract(s_vec, b, lanes)
                t_b = _extract(tot16, b, lanes)
                k_cur = k_cur - (s_b - t_b)
                t_acc = (t_acc << 4) | b

                def cc(j2, carry, cur=cur, other=other, cnt=cnt_cur,
                       shift=shift, b=b):
                    n, c2 = carry
                    for u in range(2):
                        j = j2 * 2 + u
                        kv = cur[pl.ds(j * 16, 16)]
                        valid = j < cnt
                        dg = lax.shift_right_logical(kv, shift) & 15
                        mg = valid & (dg > b)
                        pos = n + plsc.cumsum(jnp.where(mg, 1, 0)) - 1
                        val = (~kv) if invert else kv
                        plsc.store_scatter(gref, [pos], val, mask=mg)
                        n = n + plsc.all_reduce_population_count(mg)
                        if shift > 0:
                            me = valid & (dg == b)
                            plsc.store_scatter(
                                other, [c2 * 16 + lanes], kv, mask=me)
                            c2 = c2 + jnp.where(me, 1, 0)
                    return (n, c2)

                n, cnt_cur = lax.fori_loop(0, jm2, cc, (n, zeros))
                cur, other = other, cur
            return t_acc, n

        def sort256_desc(gref, out_base):
            blocks = []
            for i in range(16):
                blocks.append(_vsort_desc(gref[pl.ds(i * 16, 16)]))
            for (a, b) in _PAIRS16:
                rb = lax.rev(blocks[b], (0,))
                hi = jnp.maximum(blocks[a], rb)
                lo = jnp.minimum(blocks[a], rb)
                blocks[a] = _vsort_desc(hi)
                blocks[b] = _vsort_desc(lo)
            for i in range(16):
                s = blocks[i]
                outv[pl.ds(out_base + i * 16, 16)] = jnp.where(
                    s >= 0, s, (~s) ^ _SIGN)

        def row_body(j, _):
            row = wid * _RPW + j
            pltpu.sync_copy(in_hbm.at[row], rowf)

            def p1(i, _):
                for u in range(4):
                    base = (i * 4 + u) * 16
                    bits = rowf[pl.ds(base, 16)]
                    sra = lax.shift_right_arithmetic(bits, 31)
                    s = bits ^ lax.shift_right_logical(sra, 1)
                    keys[pl.ds(base, 16)] = s
                    d = lax.shift_right_logical(s, 24) ^ 0x80
                    plsc.addupdate_scatter(
                        hist, [u * 4096 + d * 16 + lanes], ones,
                        mask=full_mask)
                return 0

            lax.fori_loop(0, _NV // 4, p1, 0)

            bs_list = totals256()
            k = jnp.int32(_K)
            b1_t, above_t = _find_bucket256(
                lambda i: totb[pl.ds(i * 16, 16)], bs_list, k, lanes)
            k2_t = k - above_t
            b1_w, above_w = _find_bucket256(
                lambda i: lax.rev(totb[pl.ds((15 - i) * 16, 16)], (0,)),
                bs_list[::-1], k, lanes)
            k2_b = k - above_w
            b1_b = 255 - b1_w

            def p2(i, carry):
                cnt_t, cnt_b, cg_t, cg_b = carry
                for u in range(2):
                    base = (i * 2 + u) * 16
                    s = keys[pl.ds(base, 16)]
                    d = lax.shift_right_logical(s, 24) ^ 0x80
                    mt = d == b1_t
                    plsc.store_scatter(tie_t, [cnt_t * 16 + lanes], s, mask=mt)
                    cnt_t = cnt_t + jnp.where(mt, 1, 0)
                    mb = d == b1_b
                    plsc.store_scatter(tie_b, [cnt_b * 16 + lanes], ~s,
                                       mask=mb)
                    cnt_b = cnt_b + jnp.where(mb, 1, 0)
                    mgt = d > b1_t
                    plsc.store_scatter(conf_t, [cg_t * 16 + lanes], s,
                                       mask=mgt)
                    cg_t = cg_t + jnp.where(mgt, 1, 0)
                    mlb = d < b1_b
                    plsc.store_scatter(conf_b, [cg_b * 16 + lanes], s,
                                       mask=mlb)
                    cg_b = cg_b + jnp.where(mlb, 1, 0)
                return (cnt_t, cnt_b, cg_t, cg_b)

            cnt_t, cnt_b, cg_t, cg_b = lax.fori_loop(
                0, _NV // 2, p2, (zeros, zeros, zeros, zeros))

            def comp_t(j, n):
                kv = conf_t[pl.ds(j * 16, 16)]
                mk = j < cg_t
                pos = n + plsc.cumsum(jnp.where(mk, 1, 0)) - 1
                plsc.store_scatter(gtop, [pos], kv, mask=mk)
                return n + plsc.all_reduce_population_count(mk)

            ng = lax.fori_loop(0, jnp.max(cg_t), comp_t, zeros)

            def comp_b(j, n):
                kv = conf_b[pl.ds(j * 16, 16)]
                mk = j < cg_b
                pos = n + plsc.cumsum(jnp.where(mk, 1, 0)) - 1
                plsc.store_scatter(gbot, [pos], kv, mask=mk)
                return n + plsc.all_reduce_population_count(mk)

            nl = lax.fori_loop(0, jnp.max(cg_b), comp_b, zeros)

            t_top, ng = refine_collect(tie_t, cnt_t, b1_t, k2_t, gtop, ng,
                                       False)
            t_bot_w, nl = refine_collect(tie_b, cnt_b, b1_w, k2_b, gbot, nl,
                                         True)
            t_bot = ~t_bot_w

            for i in range(17):
                ids = i * 16 + lanes
                cur_t = gtop[pl.ds(i * 16, 16)]
                gtop[pl.ds(i * 16, 16)] = jnp.where(ids >= ng, t_top, cur_t)
                cur_b = gbot[pl.ds(i * 16, 16)]
                gbot[pl.ds(i * 16, 16)] = jnp.where(ids >= nl, t_bot, cur_b)

            sort256_desc(gtop, 0)
            sort256_desc(gbot, _K)

            pltpu.sync_copy(outv, out_hbm.at[row])
            return 0

        def z0(i, _):
            hist[pl.ds(i * 16, 16)] = zeros
            return 0

        lax.fori_loop(0, 1024, z0, 0)

        def z1(i, _):
            hist16[pl.ds(i * 16, 16)] = zeros
            return 0

        lax.fori_loop(0, 32, z1, 0)

        lax.fori_loop(0, _RPW, row_body, 0)

    return minmax


_minmax_kernel = _make_kernel()


@jax.jit
def kernel(input):
    x = lax.bitcast_convert_type(input.reshape(_ROWS, _ROW), jnp.int32)
    out = _minmax_kernel(x)
    return lax.bitcast_convert_type(out, jnp.float32).reshape(64, 16, 2 * _K)


# speedup vs baseline: 1.0686x; 1.0686x over previous
import functools

import jax
import jax.numpy as jnp
from jax import lax
from jax.experimental import pallas as pl
from jax.experimental.pallas import tpu as pltpu
from jax.experimental.pallas import tpu_sc as plsc

_K = 256
_L = 16
_ROWS = 1024
_ROW = 8192
_NV = _ROW // _L
_NW = 32
_RPW = _ROWS // _NW
_SIGN = -2147483648


def _batcher_pairs(n):
    pairs = []

    def merge(lo, cnt, r):
        step = r * 2
        if step < cnt:
            merge(lo, cnt, step)
            merge(lo + r, cnt, step)
            for i in range(lo + r, lo + cnt - r, step):
                pairs.append((i, i + r))
        else:
            pairs.append((lo, lo + r))

    def sort(lo, cnt):
        if cnt > 1:
            m = cnt // 2
            sort(lo, m)
            sort(lo + m, m)
            merge(lo, cnt, 1)

    sort(0, n)
    return pairs


_PAIRS16 = _batcher_pairs(16)


def _vsort_desc(v):
    ks, _ = plsc.sort_key_val(v, v, descending=True)
    return ks


def _extract(vec, idx, lanes):
    return jnp.sum(jnp.where(lanes == idx, vec, 0))


def _suffix(v):
    return lax.rev(plsc.cumsum(lax.rev(v, (0,))), (0,))


def _find_bucket256(block_vec, bs_list, k, lanes):
    sfx = [None] * 16
    run = jnp.int32(0)
    for i in range(15, -1, -1):
        run = run + bs_list[i]
        sfx[i] = run
    bidx = jnp.int32(0)
    sfx_sel = sfx[0]
    bs_sel = bs_list[0]
    for i in range(1, 16):
        c = sfx[i] >= k
        bidx = jnp.where(c, jnp.int32(i), bidx)
        sfx_sel = jnp.where(c, sfx[i], sfx_sel)
        bs_sel = jnp.where(c, bs_list[i], bs_sel)
    bt = block_vec(bidx)
    s_vec = _suffix(bt) + (sfx_sel - bs_sel)
    mask = s_vec >= k
    nge = plsc.all_reduce_population_count(mask)[0]
    boff = nge - 1
    s_b = _extract(s_vec, boff, lanes)
    t_b = _extract(bt, boff, lanes)
    return bidx * 16 + boff, s_b - t_b


def _make_kernel():
    mesh = plsc.VectorSubcoreMesh(core_axis_name="c", subcore_axis_name="s")

    @functools.partial(
        pl.kernel,
        mesh=mesh,
        compiler_params=pltpu.CompilerParams(needs_layout_passes=False),
        out_type=jax.ShapeDtypeStruct((_ROWS, _K * 2), jnp.int32),
        scratch_types=[
            pltpu.VMEM((_ROW,), jnp.int32),
            pltpu.VMEM((_ROW,), jnp.int32),
            pltpu.VMEM((_ROW,), jnp.int32),
            pltpu.VMEM((256 * _L,), jnp.int32),
            pltpu.VMEM((256,), jnp.int32),
            pltpu.VMEM((16 * _L,), jnp.int32),
            pltpu.VMEM((_ROW,), jnp.int32),
            pltpu.VMEM((_ROW,), jnp.int32),
            pltpu.VMEM((_ROW,), jnp.int32),
            pltpu.VMEM((256 * _L,), jnp.int32),
            pltpu.VMEM((256 * _L,), jnp.int32),
            pltpu.VMEM((_K + _L,), jnp.int32),
            pltpu.VMEM((_K + _L,), jnp.int32),
            pltpu.VMEM((2 * _K,), jnp.int32),
            pltpu.SemaphoreType.DMA,
            pltpu.SemaphoreType.DMA,
        ],
    )
    def minmax(in_hbm, out_hbm, rowa, rowb, keys, hist, totb, hist16, tie_t,
               tie_b, tie_s, conf_t, conf_b, gtop, gbot, outv, sema, semb):
        wid = lax.axis_index("s") * 2 + lax.axis_index("c")
        lanes = lax.iota(jnp.int32, 16)
        ones = jnp.ones((16,), jnp.int32)
        zeros = jnp.zeros((16,), jnp.int32)
        full_mask = ones > 0

        def totals256():
            def tb(i, _):
                acc = zeros
                for u in range(16):
                    off = (i * 16 + u) * 16
                    v = hist[pl.ds(off, 16)]
                    hist[pl.ds(off, 16)] = zeros
                    acc = jnp.where(lanes == u, jnp.sum(v), acc)
                totb[pl.ds(i * 16, 16)] = acc
                return 0

            lax.fori_loop(0, 16, tb, 0)
            return [jnp.sum(totb[pl.ds(i * 16, 16)]) for i in range(16)]

        def refine_collect(tie_a, cnt_vec, b1, k2, gref, n0, invert):
            k_cur = k2
            t_acc = b1 ^ 0x80
            cur, other = tie_a, tie_s
            cnt_cur = cnt_vec
            n = n0
            for shift in (20, 16, 12, 8, 4, 0):
                jm2 = (jnp.max(cnt_cur) + 1) // 2

                def hb(j2, _, cur=cur, cnt=cnt_cur, shift=shift):
                    for u in range(2):
                        j = j2 * 2 + u
                        kv = cur[pl.ds(j * 16, 16)]
                        mk = j < cnt
                        dg = lax.shift_right_logical(kv, shift) & 15
                        plsc.addupdate_scatter(
                            hist16, [dg * 16 + lanes], ones, mask=mk)
                    return 0

                lax.fori_loop(0, jm2, hb, 0)

                def tl(u, acc):
                    v = hist16[pl.ds(u * 16, 16)]
                    hist16[pl.ds(u * 16, 16)] = zeros
                    return jnp.where(lanes == u, jnp.sum(v), acc)

                tot16 = lax.fori_loop(0, 16, tl, zeros)
                s_vec = _suffix(tot16)
                mask = s_vec >= k_cur
                nge = plsc.all_reduce_population_count(mask)[0]
                b = nge - 1
                s_b = _extract(s_vec, b, lanes)
                t_b = _extract(tot16, b, lanes)
                k_cur = k_cur - (s_b - t_b)
                t_acc = (t_acc << 4) | b

                def cc(j2, carry, cur=cur, other=other, cnt=cnt_cur,
                       shift=shift, b=b):
                    n, c2 = carry
                    for u in range(2):
                        j = j2 * 2 + u
                        kv = cur[pl.ds(j * 16, 16)]
                        valid = j < cnt
                        dg = lax.shift_right_logical(kv, shift) & 15
                        mg = valid & (dg > b)
                        pos = n + plsc.cumsum(jnp.where(mg, 1, 0)) - 1
                        val = (~kv) if invert else kv
                        plsc.store_scatter(gref, [pos], val, mask=mg)
                        n = n + plsc.all_reduce_population_count(mg)
                        if shift > 0:
                            me = valid & (dg == b)
                            plsc.store_scatter(
                                other, [c2 * 16 + lanes], kv, mask=me)
                            c2 = c2 + jnp.where(me, 1, 0)
                    return (n, c2)

                n, cnt_cur = lax.fori_loop(0, jm2, cc, (n, zeros))
                cur, other = other, cur
            return t_acc, n

        def sort256_desc(gref, out_base):
            blocks = []
            for i in range(16):
                blocks.append(_vsort_desc(gref[pl.ds(i * 16, 16)]))
            for (a, b) in _PAIRS16:
                rb = lax.rev(blocks[b], (0,))
                hi = jnp.maximum(blocks[a], rb)
                lo = jnp.minimum(blocks[a], rb)
                blocks[a] = _vsort_desc(hi)
                blocks[b] = _vsort_desc(lo)
            for i in range(16):
                s = blocks[i]
                outv[pl.ds(out_base + i * 16, 16)] = jnp.where(
                    s >= 0, s, (~s) ^ _SIGN)

        def row_body(row, rowf):
            def p1(i, _):
                for u in range(4):
                    base = (i * 4 + u) * 16
                    bits = rowf[pl.ds(base, 16)]
                    sra = lax.shift_right_arithmetic(bits, 31)
                    s = bits ^ lax.shift_right_logical(sra, 1)
                    keys[pl.ds(base, 16)] = s
                    d = lax.shift_right_logical(s, 24) ^ 0x80
                    plsc.addupdate_scatter(
                        hist, [d * 16 + lanes], ones, mask=full_mask)
                return 0

            lax.fori_loop(0, _NV // 4, p1, 0)

            bs_list = totals256()
            k = jnp.int32(_K)
            b1_t, above_t = _find_bucket256(
                lambda i: totb[pl.ds(i * 16, 16)], bs_list, k, lanes)
            k2_t = k - above_t
            b1_w, above_w = _find_bucket256(
                lambda i: lax.rev(totb[pl.ds((15 - i) * 16, 16)], (0,)),
                bs_list[::-1], k, lanes)
            k2_b = k - above_w
            b1_b = 255 - b1_w

            def p2(i, carry):
                cnt_t, cnt_b, cg_t, cg_b = carry
                for u in range(2):
                    base = (i * 2 + u) * 16
                    s = keys[pl.ds(base, 16)]
                    d = lax.shift_right_logical(s, 24) ^ 0x80
                    mt = d == b1_t
                    plsc.store_scatter(tie_t, [cnt_t * 16 + lanes], s, mask=mt)
                    cnt_t = cnt_t + jnp.where(mt, 1, 0)
                    mb = d == b1_b
                    plsc.store_scatter(tie_b, [cnt_b * 16 + lanes], ~s,
                                       mask=mb)
                    cnt_b = cnt_b + jnp.where(mb, 1, 0)
                    mgt = d > b1_t
                    plsc.store_scatter(conf_t, [cg_t * 16 + lanes], s,
                                       mask=mgt)
                    cg_t = cg_t + jnp.where(mgt, 1, 0)
                    mlb = d < b1_b
                    plsc.store_scatter(conf_b, [cg_b * 16 + lanes], s,
                                       mask=mlb)
                    cg_b = cg_b + jnp.where(mlb, 1, 0)
                return (cnt_t, cnt_b, cg_t, cg_b)

            cnt_t, cnt_b, cg_t, cg_b = lax.fori_loop(
                0, _NV // 2, p2, (zeros, zeros, zeros, zeros))

            def comp_t(j, n):
                kv = conf_t[pl.ds(j * 16, 16)]
                mk = j < cg_t
                pos = n + plsc.cumsum(jnp.where(mk, 1, 0)) - 1
                plsc.store_scatter(gtop, [pos], kv, mask=mk)
                return n + plsc.all_reduce_population_count(mk)

            ng = lax.fori_loop(0, jnp.max(cg_t), comp_t, zeros)

            def comp_b(j, n):
                kv = conf_b[pl.ds(j * 16, 16)]
                mk = j < cg_b
                pos = n + plsc.cumsum(jnp.where(mk, 1, 0)) - 1
                plsc.store_scatter(gbot, [pos], kv, mask=mk)
                return n + plsc.all_reduce_population_count(mk)

            nl = lax.fori_loop(0, jnp.max(cg_b), comp_b, zeros)

            t_top, ng = refine_collect(tie_t, cnt_t, b1_t, k2_t, gtop, ng,
                                       False)
            t_bot_w, nl = refine_collect(tie_b, cnt_b, b1_w, k2_b, gbot, nl,
                                         True)
            t_bot = ~t_bot_w

            for i in range(17):
                ids = i * 16 + lanes
                cur_t = gtop[pl.ds(i * 16, 16)]
                gtop[pl.ds(i * 16, 16)] = jnp.where(ids >= ng, t_top, cur_t)
                cur_b = gbot[pl.ds(i * 16, 16)]
                gbot[pl.ds(i * 16, 16)] = jnp.where(ids >= nl, t_bot, cur_b)

            sort256_desc(gtop, 0)
            sort256_desc(gbot, _K)

            pltpu.sync_copy(outv, out_hbm.at[row])

        def z0(i, _):
            hist[pl.ds(i * 16, 16)] = zeros
            return 0

        lax.fori_loop(0, 256, z0, 0)

        def z1(i, _):
            hist16[pl.ds(i * 16, 16)] = zeros
            return 0

        lax.fori_loop(0, 16, z1, 0)

        first = wid * _RPW
        pltpu.async_copy(in_hbm.at[first], rowa, sema)

        def row_pair(jj, _):
            r0 = first + jj * 2
            pltpu.make_async_copy(in_hbm.at[0], rowa, sema).wait()
            pltpu.async_copy(in_hbm.at[r0 + 1], rowb, semb)
            row_body(r0, rowa)
            pltpu.make_async_copy(in_hbm.at[0], rowb, semb).wait()
            nxt = jnp.minimum(r0 + 2, first + _RPW - 1)
            pltpu.async_copy(in_hbm.at[nxt], rowa, sema)
            row_body(r0 + 1, rowb)
            return 0

        lax.fori_loop(0, _RPW // 2, row_pair, 0)
        pltpu.make_async_copy(in_hbm.at[0], rowa, sema).wait()

    return minmax


_minmax_kernel = _make_kernel()


@jax.jit
def kernel(input):
    x = lax.bitcast_convert_type(input.reshape(_ROWS, _ROW), jnp.int32)
    out = _minmax_kernel(x)
    return lax.bitcast_convert_type(out, jnp.float32).reshape(64, 16, 2 * _K)


# speedup vs baseline: 1.0718x; 1.0030x over previous
import functools

import jax
import jax.numpy as jnp
from jax import lax
from jax.experimental import pallas as pl
from jax.experimental.pallas import tpu as pltpu
from jax.experimental.pallas import tpu_sc as plsc

_K = 256
_L = 16
_ROWS = 1024
_ROW = 8192
_NV = _ROW // _L
_NW = 32
_RPW = _ROWS // _NW
_SIGN = -2147483648


def _batcher_pairs(n):
    pairs = []

    def merge(lo, cnt, r):
        step = r * 2
        if step < cnt:
            merge(lo, cnt, step)
            merge(lo + r, cnt, step)
            for i in range(lo + r, lo + cnt - r, step):
                pairs.append((i, i + r))
        else:
            pairs.append((lo, lo + r))

    def sort(lo, cnt):
        if cnt > 1:
            m = cnt // 2
            sort(lo, m)
            sort(lo + m, m)
            merge(lo, cnt, 1)

    sort(0, n)
    return pairs


_PAIRS16 = _batcher_pairs(16)


def _vsort_desc(v):
    ks, _ = plsc.sort_key_val(v, v, descending=True)
    return ks


def _extract(vec, idx, lanes):
    return jnp.sum(jnp.where(lanes == idx, vec, 0))


def _suffix(v):
    return lax.rev(plsc.cumsum(lax.rev(v, (0,))), (0,))


def _find_bucket256(block_vec, bs_list, k, lanes):
    sfx = [None] * 16
    run = jnp.int32(0)
    for i in range(15, -1, -1):
        run = run + bs_list[i]
        sfx[i] = run
    bidx = jnp.int32(0)
    sfx_sel = sfx[0]
    bs_sel = bs_list[0]
    for i in range(1, 16):
        c = sfx[i] >= k
        bidx = jnp.where(c, jnp.int32(i), bidx)
        sfx_sel = jnp.where(c, sfx[i], sfx_sel)
        bs_sel = jnp.where(c, bs_list[i], bs_sel)
    bt = block_vec(bidx)
    s_vec = _suffix(bt) + (sfx_sel - bs_sel)
    mask = s_vec >= k
    nge = plsc.all_reduce_population_count(mask)[0]
    boff = nge - 1
    s_b = _extract(s_vec, boff, lanes)
    t_b = _extract(bt, boff, lanes)
    return bidx * 16 + boff, s_b - t_b


def _make_kernel():
    mesh = plsc.VectorSubcoreMesh(core_axis_name="c", subcore_axis_name="s")

    @functools.partial(
        pl.kernel,
        mesh=mesh,
        compiler_params=pltpu.CompilerParams(needs_layout_passes=False),
        out_type=jax.ShapeDtypeStruct((_ROWS, _K * 2), jnp.int32),
        scratch_types=[
            pltpu.VMEM((_ROW,), jnp.int32),
            pltpu.VMEM((_ROW,), jnp.int32),
            pltpu.VMEM((_ROW,), jnp.int32),
            pltpu.VMEM((256 * _L,), jnp.int32),
            pltpu.VMEM((256,), jnp.int32),
            pltpu.VMEM((16 * _L,), jnp.int32),
            pltpu.VMEM((_ROW,), jnp.int32),
            pltpu.VMEM((_ROW,), jnp.int32),
            pltpu.VMEM((_ROW,), jnp.int32),
            pltpu.VMEM((256 * _L,), jnp.int32),
            pltpu.VMEM((256 * _L,), jnp.int32),
            pltpu.VMEM((_K + _L,), jnp.int32),
            pltpu.VMEM((_K + _L,), jnp.int32),
            pltpu.VMEM((2 * _K,), jnp.int32),
            pltpu.SemaphoreType.DMA,
            pltpu.SemaphoreType.DMA,
        ],
    )
    def minmax(in_hbm, out_hbm, rowa, rowb, keys, hist, totb, hist16, tie_t,
               tie_b, tie_s, conf_t, conf_b, gtop, gbot, outv, sema, semb):
        wid = lax.axis_index("s") * 2 + lax.axis_index("c")
        lanes = lax.iota(jnp.int32, 16)
        ones = jnp.ones((16,), jnp.int32)
        zeros = jnp.zeros((16,), jnp.int32)
        full_mask = ones > 0

        def totals256():
            def tb(i, _):
                acc = zeros
                for u in range(16):
                    off = (i * 16 + u) * 16
                    v = hist[pl.ds(off, 16)]
                    hist[pl.ds(off, 16)] = zeros
                    acc = jnp.where(lanes == u, jnp.sum(v), acc)
                totb[pl.ds(i * 16, 16)] = acc
                return 0

            lax.fori_loop(0, 16, tb, 0)
            return [jnp.sum(totb[pl.ds(i * 16, 16)]) for i in range(16)]

        def refine_collect(tie_a, cnt_vec, b1, k2, gref, n0, invert):
            k_cur = k2
            t_acc = b1 ^ 0x80
            cur, other = tie_a, tie_s
            cnt_cur = cnt_vec
            n = n0
            for shift in (20, 16, 12, 8, 4, 0):
                jm2 = (jnp.max(cnt_cur) + 1) // 2

                def hb(j2, _, cur=cur, cnt=cnt_cur, shift=shift):
                    for u in range(2):
                        j = j2 * 2 + u
                        kv = cur[pl.ds(j * 16, 16)]
                        mk = j < cnt
                        dg = lax.shift_right_logical(kv, shift) & 15
                        plsc.addupdate_scatter(
                            hist16, [dg * 16 + lanes], ones, mask=mk)
                    return 0

                lax.fori_loop(0, jm2, hb, 0)

                def tl(u, acc):
                    v = hist16[pl.ds(u * 16, 16)]
                    hist16[pl.ds(u * 16, 16)] = zeros
                    return jnp.where(lanes == u, jnp.sum(v), acc)

                tot16 = lax.fori_loop(0, 16, tl, zeros)
                s_vec = _suffix(tot16)
                mask = s_vec >= k_cur
                nge = plsc.all_reduce_population_count(mask)[0]
                b = nge - 1
                s_b = _extract(s_vec, b, lanes)
                t_b = _extract(tot16, b, lanes)
                k_cur = k_cur - (s_b - t_b)
                t_acc = (t_acc << 4) | b

                def cc(j2, carry, cur=cur, other=other, cnt=cnt_cur,
                       shift=shift, b=b):
                    n, c2 = carry
                    for u in range(2):
                        j = j2 * 2 + u
                        kv = cur[pl.ds(j * 16, 16)]
                        valid = j < cnt
                        dg = lax.shift_right_logical(kv, shift) & 15
                        mg = valid & (dg > b)
                        pos = n + plsc.cumsum(jnp.where(mg, 1, 0)) - 1
                        val = (~kv) if invert else kv
                        plsc.store_scatter(gref, [pos], val, mask=mg)
                        n = n + plsc.all_reduce_population_count(mg)
                        if shift > 0:
                            me = valid & (dg == b)
                            plsc.store_scatter(
                                other, [c2 * 16 + lanes], kv, mask=me)
                            c2 = c2 + jnp.where(me, 1, 0)
                    return (n, c2)

                n, cnt_cur = lax.fori_loop(0, jm2, cc, (n, zeros))
                cur, other = other, cur
            return t_acc, n

        def sort256_desc(gref, out_base):
            blocks = []
            for i in range(16):
                blocks.append(_vsort_desc(gref[pl.ds(i * 16, 16)]))
            for (a, b) in _PAIRS16:
                rb = lax.rev(blocks[b], (0,))
                hi = jnp.maximum(blocks[a], rb)
                lo = jnp.minimum(blocks[a], rb)
                blocks[a] = _vsort_desc(hi)
                blocks[b] = _vsort_desc(lo)
            for i in range(16):
                s = blocks[i]
                outv[pl.ds(out_base + i * 16, 16)] = jnp.where(
                    s >= 0, s, (~s) ^ _SIGN)

        def row_body(row, rowf):
            def p1(i, _):
                for u in range(4):
                    base = (i * 4 + u) * 16
                    bits = rowf[pl.ds(base, 16)]
                    sra = lax.shift_right_arithmetic(bits, 31)
                    s = bits ^ lax.shift_right_logical(sra, 1)
                    keys[pl.ds(base, 16)] = s
                    d = lax.shift_right_logical(s, 24) ^ 0x80
                    plsc.addupdate_scatter(
                        hist, [d * 16 + lanes], ones, mask=full_mask)
                return 0

            lax.fori_loop(0, _NV // 4, p1, 0)

            bs_list = totals256()
            k = jnp.int32(_K)
            b1_t, above_t = _find_bucket256(
                lambda i: totb[pl.ds(i * 16, 16)], bs_list, k, lanes)
            k2_t = k - above_t
            b1_w, above_w = _find_bucket256(
                lambda i: lax.rev(totb[pl.ds((15 - i) * 16, 16)], (0,)),
                bs_list[::-1], k, lanes)
            k2_b = k - above_w
            b1_b = 255 - b1_w

            def p2(i, carry):
                cnt_t, cnt_b, cg_t, cg_b = carry
                for u in range(4):
                    base = (i * 4 + u) * 16
                    s = keys[pl.ds(base, 16)]
                    d = lax.shift_right_logical(s, 24) ^ 0x80
                    mt = d == b1_t
                    plsc.store_scatter(tie_t, [cnt_t], s, mask=mt)
                    cnt_t = cnt_t + jnp.where(mt, 16, 0)
                    mb = d == b1_b
                    plsc.store_scatter(tie_b, [cnt_b], ~s, mask=mb)
                    cnt_b = cnt_b + jnp.where(mb, 16, 0)
                    mgt = d > b1_t
                    plsc.store_scatter(conf_t, [cg_t], s, mask=mgt)
                    cg_t = cg_t + jnp.where(mgt, 16, 0)
                    mlb = d < b1_b
                    plsc.store_scatter(conf_b, [cg_b], s, mask=mlb)
                    cg_b = cg_b + jnp.where(mlb, 16, 0)
                return (cnt_t, cnt_b, cg_t, cg_b)

            cnt_t, cnt_b, cg_t, cg_b = lax.fori_loop(
                0, _NV // 4, p2, (lanes, lanes, lanes, lanes))
            unscale = lambda c: lax.shift_right_logical(c - lanes, 4)
            cnt_t, cnt_b = unscale(cnt_t), unscale(cnt_b)
            cg_t, cg_b = unscale(cg_t), unscale(cg_b)

            def comp_t(j, n):
                kv = conf_t[pl.ds(j * 16, 16)]
                mk = j < cg_t
                pos = n + plsc.cumsum(jnp.where(mk, 1, 0)) - 1
                plsc.store_scatter(gtop, [pos], kv, mask=mk)
                return n + plsc.all_reduce_population_count(mk)

            ng = lax.fori_loop(0, jnp.max(cg_t), comp_t, zeros)

            def comp_b(j, n):
                kv = conf_b[pl.ds(j * 16, 16)]
                mk = j < cg_b
                pos = n + plsc.cumsum(jnp.where(mk, 1, 0)) - 1
                plsc.store_scatter(gbot, [pos], kv, mask=mk)
                return n + plsc.all_reduce_population_count(mk)

            nl = lax.fori_loop(0, jnp.max(cg_b), comp_b, zeros)

            t_top, ng = refine_collect(tie_t, cnt_t, b1_t, k2_t, gtop, ng,
                                       False)
            t_bot_w, nl = refine_collect(tie_b, cnt_b, b1_w, k2_b, gbot, nl,
                                         True)
            t_bot = ~t_bot_w

            for i in range(17):
                ids = i * 16 + lanes
                cur_t = gtop[pl.ds(i * 16, 16)]
                gtop[pl.ds(i * 16, 16)] = jnp.where(ids >= ng, t_top, cur_t)
                cur_b = gbot[pl.ds(i * 16, 16)]
                gbot[pl.ds(i * 16, 16)] = jnp.where(ids >= nl, t_bot, cur_b)

            sort256_desc(gtop, 0)
            sort256_desc(gbot, _K)

            pltpu.sync_copy(outv, out_hbm.at[row])

        def z0(i, _):
            hist[pl.ds(i * 16, 16)] = zeros
            return 0

        lax.fori_loop(0, 256, z0, 0)

        def z1(i, _):
            hist16[pl.ds(i * 16, 16)] = zeros
            return 0

        lax.fori_loop(0, 16, z1, 0)

        first = wid * _RPW
        pltpu.async_copy(in_hbm.at[first], rowa, sema)

        def row_pair(jj, _):
            r0 = first + jj * 2
            pltpu.make_async_copy(in_hbm.at[0], rowa, sema).wait()
            pltpu.async_copy(in_hbm.at[r0 + 1], rowb, semb)
            row_body(r0, rowa)
            pltpu.make_async_copy(in_hbm.at[0], rowb, semb).wait()
            nxt = jnp.minimum(r0 + 2, first + _RPW - 1)
            pltpu.async_copy(in_hbm.at[nxt], rowa, sema)
            row_body(r0 + 1, rowb)
            return 0

        lax.fori_loop(0, _RPW // 2, row_pair, 0)
        pltpu.make_async_copy(in_hbm.at[0], rowa, sema).wait()

    return minmax


_minmax_kernel = _make_kernel()


@jax.jit
def kernel(input):
    x = lax.bitcast_convert_type(input.reshape(_ROWS, _ROW), jnp.int32)
    out = _minmax_kernel(x)
    return lax.bitcast_convert_type(out, jnp.float32).reshape(64, 16, 2 * _K)
